# Initial kernel scaffold; baseline (speedup 1.0000x reference)
import jax
import jax.numpy as jnp
from jax.experimental import pallas as pl


def kernel(x_field, x, W, bias, offsets):
    raise NotImplementedError("write your pallas kernel here")


# speedup vs baseline: 1.7820x; 1.7820x over previous
import functools

import jax
import jax.numpy as jnp
from jax import lax
from jax.experimental import pallas as pl
from jax.experimental.pallas import tpu as pltpu
from jax.experimental.pallas import tpu_sc as plsc

_NUM_FIELDS = 26
_BATCH = 4096
_LANES = 16

_info = plsc.get_sparse_core_info()
_NC, _NS = _info.num_cores, _info.num_subcores
_NW = _NC * _NS
_BPW = _BATCH // _NW
_IDX_PER_W = _BPW * _NUM_FIELDS
_N_CHUNKS = _IDX_PER_W // _LANES

_DIV26_MUL = 40330
_DIV26_SHIFT = 20

_mesh = plsc.VectorSubcoreMesh(core_axis_name="c", subcore_axis_name="s")


@functools.partial(
    pl.kernel,
    mesh=_mesh,
    compiler_params=pltpu.CompilerParams(
        use_tc_tiling_on_sc=False, needs_layout_passes=False),
    out_type=jax.ShapeDtypeStruct((_BATCH,), jnp.float32),
    scratch_types=[
        pltpu.VMEM((_IDX_PER_W,), jnp.int32),
        pltpu.VMEM((_IDX_PER_W,), jnp.int32),
        pltpu.VMEM((32,), jnp.int32),
        pltpu.VMEM((_NUM_FIELDS, _BPW), jnp.int32),
        pltpu.VMEM((_NUM_FIELDS, _BPW), jnp.float32),
        pltpu.VMEM((_BPW,), jnp.float32),
        pltpu.SemaphoreType.DMA,
    ],
)
def _features_linear_sc(x_hbm, xf_hbm, w_hbm, off_hbm, out_hbm,
                        x_v, xf_v, off_v, idx_v, rows_v, out_v, sem):
    wid = lax.axis_index("s") * _NC + lax.axis_index("c")
    base = wid * _BPW

    pltpu.sync_copy(x_hbm.at[pl.ds(base * _NUM_FIELDS, _IDX_PER_W)], x_v)
    pltpu.sync_copy(xf_hbm.at[pl.ds(base * _NUM_FIELDS, _IDX_PER_W)], xf_v)
    pltpu.sync_copy(off_hbm, off_v)

    def idx_body(i, carry):
        sl = pl.ds(i * _LANES, _LANES)
        p = lax.iota(jnp.int32, _LANES) + i * _LANES
        b = lax.shift_right_logical(p * _DIV26_MUL, _DIV26_SHIFT)
        f = p - b * _NUM_FIELDS
        off = plsc.load_gather(off_v, [xf_v[sl]])
        plsc.store_scatter(idx_v, [f, b], x_v[sl] + off)
        return carry

    lax.fori_loop(0, _N_CHUNKS, idx_body, 0)

    for c in range(_NUM_FIELDS):
        pltpu.make_async_copy(w_hbm.at[idx_v.at[c]], rows_v.at[c], sem).start()
    for c in range(_NUM_FIELDS):
        pltpu.make_async_copy(w_hbm.at[idx_v.at[c]], rows_v.at[c], sem).wait()

    def red_body(bc, carry):
        sl = pl.ds(bc * _LANES, _LANES)
        acc = rows_v[0, sl]
        for c in range(1, _NUM_FIELDS):
            acc = acc + rows_v[c, sl]
        out_v[sl] = acc
        return carry

    lax.fori_loop(0, _BPW // _LANES, red_body, 0)

    pltpu.sync_copy(out_v, out_hbm.at[pl.ds(base, _BPW)])


def kernel(x_field, x, W, bias, offsets):
    x_flat = x.reshape(-1)
    xf_flat = x_field.reshape(-1)
    w_flat = W.reshape(-1)
    off_pad = jnp.pad(offsets, (0, 32 - offsets.shape[0]))
    out = _features_linear_sc(x_flat, xf_flat, w_flat, off_pad)
    return out.reshape(_BATCH, 1) + bias


# speedup vs baseline: 1.8895x; 1.0603x over previous
import functools

import jax
import jax.numpy as jnp
from jax import lax
from jax.experimental import pallas as pl
from jax.experimental.pallas import tpu as pltpu
from jax.experimental.pallas import tpu_sc as plsc

_NUM_FIELDS = 26
_BATCH = 4096
_LANES = 16
_PACK_SHIFT = 5
_PACK_MASK = 31

_info = plsc.get_sparse_core_info()
_NC, _NS = _info.num_cores, _info.num_subcores
_NW = _NC * _NS
_BPW = _BATCH // _NW
_IDX_PER_W = _BPW * _NUM_FIELDS

_mesh = plsc.VectorSubcoreMesh(core_axis_name="c", subcore_axis_name="s")


@functools.partial(
    pl.kernel,
    mesh=_mesh,
    compiler_params=pltpu.CompilerParams(
        use_tc_tiling_on_sc=False, needs_layout_passes=False),
    out_type=jax.ShapeDtypeStruct((_BATCH,), jnp.float32),
    scratch_types=[
        pltpu.VMEM((_IDX_PER_W,), jnp.int32),
        pltpu.VMEM((32,), jnp.int32),
        pltpu.VMEM((_NUM_FIELDS, _BPW), jnp.int32),
        pltpu.VMEM((_NUM_FIELDS, _BPW), jnp.float32),
        pltpu.VMEM((_BPW,), jnp.float32),
        pltpu.SemaphoreType.DMA,
    ],
)
def _features_linear_sc(xc_hbm, w_hbm, off_hbm, out_hbm,
                        xc_v, off_v, idx_v, rows_v, out_v, sem):
    wid = lax.axis_index("s") * _NC + lax.axis_index("c")
    base = wid * _BPW

    pltpu.sync_copy(xc_hbm.at[pl.ds(base * _NUM_FIELDS, _IDX_PER_W)], xc_v)
    pltpu.sync_copy(off_hbm, off_v)

    def idx_body(bc, carry):
        bvec = (lax.iota(jnp.int32, _LANES) + bc * _LANES) * _NUM_FIELDS
        sl = pl.ds(bc * _LANES, _LANES)
        for f in range(_NUM_FIELDS):
            xcv = plsc.load_gather(xc_v, [bvec + f])
            off = plsc.load_gather(off_v, [lax.bitwise_and(xcv, _PACK_MASK)])
            idx_v[f, sl] = lax.shift_right_logical(xcv, _PACK_SHIFT) + off
        return carry

    lax.fori_loop(0, _BPW // _LANES, idx_body, 0)

    for c in range(_NUM_FIELDS):
        pltpu.make_async_copy(w_hbm.at[idx_v.at[c]], rows_v.at[c], sem).start()
    for c in range(_NUM_FIELDS):
        pltpu.make_async_copy(w_hbm.at[idx_v.at[c]], rows_v.at[c], sem).wait()

    def red_body(bc, carry):
        sl = pl.ds(bc * _LANES, _LANES)
        acc = rows_v[0, sl]
        for c in range(1, _NUM_FIELDS):
            acc = acc + rows_v[c, sl]
        out_v[sl] = acc
        return carry

    lax.fori_loop(0, _BPW // _LANES, red_body, 0)

    pltpu.sync_copy(out_v, out_hbm.at[pl.ds(base, _BPW)])


def kernel(x_field, x, W, bias, offsets):
    xc = (x * (_PACK_MASK + 1) + x_field).reshape(-1)
    off_pad = jnp.pad(offsets, (0, 32 - offsets.shape[0]))
    out = _features_linear_sc(xc, W.reshape(-1), off_pad)
    return out.reshape(_BATCH, 1) + bias
